# baseline (speedup 1.0000x reference)
import jax
import jax.numpy as jnp
from jax.experimental import pallas as pl

BT = 512


def _add_pos_kernel(x_ref, pos_ref, out_ref):
    p = pos_ref[...]
    D = p.shape[1]
    for b in range(x_ref.shape[1] // D):
        sl = pl.ds(b * D, D)
        out_ref[:, sl] = x_ref[:, sl] + p


def kernel(x, pos_table):
    T, B, D = x.shape
    x2 = x.reshape(T, B * D)
    out2 = pl.pallas_call(
        _add_pos_kernel,
        grid=(T // BT,),
        in_specs=[
            pl.BlockSpec((BT, B * D), lambda i: (i, 0)),
            pl.BlockSpec((BT, D), lambda i: (i, 0)),
        ],
        out_specs=pl.BlockSpec((BT, B * D), lambda i: (i, 0)),
        out_shape=jax.ShapeDtypeStruct((T, B * D), x.dtype),
    )(x2, pos_table)
    return out2.reshape(T, B, D)


# speedup vs baseline: 1.5829x; 1.5829x over previous
import functools

import jax
import jax.numpy as jnp
from jax import lax
from jax.experimental import pallas as pl
from jax.experimental.pallas import tpu as pltpu
from jax.experimental.pallas import tpu_sc as plsc

T, B, D = 2048, 4, 1024
NC, NS, L = 2, 16, 16
NW = NC * NS
TPW = T // NW
CHUNK = 8
NCHUNK = TPW // CHUNK
VECS = D // L


def _sc_body(x_hbm, pos_hbm, out_hbm, x_v, pos_v, in_sems, out_sems):
    wid = lax.axis_index("s") * NC + lax.axis_index("c")
    base = wid * TPW

    def start_in(c, slot):
        t0 = base + c * CHUNK
        pltpu.async_copy(x_hbm.at[pl.ds(t0, CHUNK)], x_v.at[slot], in_sems.at[slot])
        pltpu.async_copy(pos_hbm.at[pl.ds(t0, CHUNK)], pos_v.at[slot], in_sems.at[slot])

    def wait_in(slot):
        pltpu.make_async_copy(x_hbm.at[pl.ds(0, CHUNK)], x_v.at[slot], in_sems.at[slot]).wait()
        pltpu.make_async_copy(pos_hbm.at[pl.ds(0, CHUNK)], pos_v.at[slot], in_sems.at[slot]).wait()

    def start_out(c, slot):
        t0 = base + c * CHUNK
        pltpu.async_copy(x_v.at[slot], out_hbm.at[pl.ds(t0, CHUNK)], out_sems.at[slot])

    def wait_out(slot):
        pltpu.make_async_copy(x_v.at[slot], out_hbm.at[pl.ds(0, CHUNK)], out_sems.at[slot]).wait()

    def compute(slot):
        def row_body(t, carry):
            for j in range(VECS):
                p = pos_v[slot, t, pl.ds(j * L, L)]
                for b in range(B):
                    x_v[slot, t, b, pl.ds(j * L, L)] = (
                        x_v[slot, t, b, pl.ds(j * L, L)] + p
                    )
            return carry

        lax.fori_loop(0, CHUNK, row_body, 0)

    start_in(0, 0)
    for c in range(NCHUNK):
        slot = c % 2
        if c + 1 < NCHUNK:
            if c >= 1:
                wait_out(1 - slot)
            start_in(c + 1, 1 - slot)
        wait_in(slot)
        compute(slot)
        start_out(c, slot)
    wait_out(0)
    wait_out(1)


def kernel(x, pos_table):
    mesh = plsc.VectorSubcoreMesh(core_axis_name="c", subcore_axis_name="s")
    k = functools.partial(
        pl.kernel,
        mesh=mesh,
        out_type=jax.ShapeDtypeStruct((T, B, D), jnp.float32),
        scratch_types=[
            pltpu.VMEM((2, CHUNK, B, D), jnp.float32),
            pltpu.VMEM((2, CHUNK, D), jnp.float32),
            pltpu.SemaphoreType.DMA((2,)),
            pltpu.SemaphoreType.DMA((2,)),
        ],
    )(_sc_body)
    return k(x, pos_table)


# speedup vs baseline: 2.1780x; 1.3760x over previous
import functools

import jax
import jax.numpy as jnp
from jax import lax
from jax.experimental import pallas as pl
from jax.experimental.pallas import tpu as pltpu
from jax.experimental.pallas import tpu_sc as plsc

T, B, D = 2048, 4, 1024
NC, NS, L = 2, 16, 16
NW = NC * NS
T_SC = 512
T_TC = T - T_SC
RPW = T_SC // NW
CHUNK = 8
NCHUNK = RPW // CHUNK
VECS = D // L
BT = 512


def _tc_body(x_ref, pos_ref, out_ref):
    out_ref[...] = x_ref[...] + pos_ref[...][:, None, :]


def _merge_body(sc_ref, full_ref, out_ref):
    del full_ref
    out_ref[...] = sc_ref[...]


def _sc_body(x_hbm, pos_hbm, out_hbm, x_v, pos_v, in_sems, out_sems):
    wid = lax.axis_index("s") * NC + lax.axis_index("c")
    src_base = T_TC + wid * RPW
    dst_base = wid * RPW

    def start_in(c, slot):
        t0 = src_base + c * CHUNK
        pltpu.async_copy(x_hbm.at[pl.ds(t0, CHUNK)], x_v.at[slot], in_sems.at[slot])
        pltpu.async_copy(pos_hbm.at[pl.ds(t0, CHUNK)], pos_v.at[slot], in_sems.at[slot])

    def wait_in(slot):
        pltpu.make_async_copy(x_hbm.at[pl.ds(0, CHUNK)], x_v.at[slot], in_sems.at[slot]).wait()
        pltpu.make_async_copy(pos_hbm.at[pl.ds(0, CHUNK)], pos_v.at[slot], in_sems.at[slot]).wait()

    def start_out(c, slot):
        t0 = dst_base + c * CHUNK
        pltpu.async_copy(x_v.at[slot], out_hbm.at[pl.ds(t0, CHUNK)], out_sems.at[slot])

    def wait_out(slot):
        pltpu.make_async_copy(x_v.at[slot], out_hbm.at[pl.ds(0, CHUNK)], out_sems.at[slot]).wait()

    def compute(slot):
        def row_body(t, carry):
            for j in range(VECS):
                p = pos_v[slot, t, pl.ds(j * L, L)]
                for b in range(B):
                    x_v[slot, t, b, pl.ds(j * L, L)] = (
                        x_v[slot, t, b, pl.ds(j * L, L)] + p
                    )
            return carry

        lax.fori_loop(0, CHUNK, row_body, 0)

    start_in(0, 0)
    for c in range(NCHUNK):
        slot = c % 2
        if c + 1 < NCHUNK:
            if c >= 1:
                wait_out(1 - slot)
            start_in(c + 1, 1 - slot)
        wait_in(slot)
        compute(slot)
        start_out(c, slot)
    for slot in range(min(2, NCHUNK)):
        wait_out(slot)


def kernel(x, pos_table):
    mesh = plsc.VectorSubcoreMesh(core_axis_name="c", subcore_axis_name="s")
    sc_k = functools.partial(
        pl.kernel,
        mesh=mesh,
        out_type=jax.ShapeDtypeStruct((T_SC, B, D), jnp.float32),
        scratch_types=[
            pltpu.VMEM((2, CHUNK, B, D), jnp.float32),
            pltpu.VMEM((2, CHUNK, D), jnp.float32),
            pltpu.SemaphoreType.DMA((2,)),
            pltpu.SemaphoreType.DMA((2,)),
        ],
    )(_sc_body)
    sc_out = sc_k(x, pos_table)

    full = pl.pallas_call(
        _tc_body,
        grid=(T_TC // BT,),
        in_specs=[
            pl.BlockSpec((BT, B, D), lambda i: (i, 0, 0)),
            pl.BlockSpec((BT, D), lambda i: (i, 0)),
        ],
        out_specs=pl.BlockSpec((BT, B, D), lambda i: (i, 0, 0)),
        out_shape=jax.ShapeDtypeStruct((T, B, D), x.dtype),
    )(x, pos_table)

    return pl.pallas_call(
        _merge_body,
        grid=(1,),
        in_specs=[
            pl.BlockSpec((T_SC, B, D), lambda i: (0, 0, 0)),
            pl.BlockSpec(memory_space=pl.ANY),
        ],
        out_specs=pl.BlockSpec((T_SC, B, D), lambda i: (T_TC // T_SC, 0, 0)),
        out_shape=jax.ShapeDtypeStruct((T, B, D), x.dtype),
        input_output_aliases={1: 0},
    )(sc_out, full)


# speedup vs baseline: 4.2157x; 1.9356x over previous
import jax
import jax.numpy as jnp
from jax.experimental import pallas as pl

BT = 512


def _add_pos_kernel(x_ref, pos_ref, out_ref):
    out_ref[...] = x_ref[...] + pos_ref[...][:, None, :]


def kernel(x, pos_table):
    T, B, D = x.shape
    grid = (T // BT,)
    return pl.pallas_call(
        _add_pos_kernel,
        grid=grid,
        in_specs=[
            pl.BlockSpec((BT, B, D), lambda i: (i, 0, 0)),
            pl.BlockSpec((BT, D), lambda i: (i, 0)),
        ],
        out_specs=pl.BlockSpec((BT, B, D), lambda i: (i, 0, 0)),
        out_shape=jax.ShapeDtypeStruct((T, B, D), x.dtype),
    )(x, pos_table)
